# Initial kernel scaffold; baseline (speedup 1.0000x reference)
import jax
import jax.numpy as jnp
from jax.experimental import pallas as pl


def kernel(positions, numbers, edge_index, rcov, r4r2, c6, cn_ref):
    raise NotImplementedError("write your pallas kernel here")


# speedup vs baseline: 81.1511x; 81.1511x over previous
import functools

import jax
import jax.numpy as jnp
from jax import lax
from jax.experimental import pallas as pl
from jax.experimental.pallas import tpu as pltpu
from jax.experimental.pallas import tpu_sc as plsc

N = 100000
E = 1600000
Z = 95
M = 5
A1 = 0.4
A2 = 4.8
S6 = 1.0
S8 = 1.0
CUTOFF = 50.0
CN_CUTOFF = 25.0
K1 = 16.0
K3 = 4.0

NC = 2
NS = 16
NW = NC * NS

NPAD = 102400
TN = NPAD // NW
CB = 2048
ROWS = CB // 128
NCHUNK = 25
EPAD = NW * NCHUNK * CB
TE = EPAD // NW
ZZ = Z * Z

_mesh = plsc.VectorSubcoreMesh(core_axis_name="c", subcore_axis_name="s")


def _iota16():
    return lax.iota(jnp.int32, 16)


def _full16(v):
    return jnp.full((16,), v, jnp.int32)


def _rsqrt(x):
    i = plsc.bitcast(x, jnp.int32)
    i = jnp.int32(0x5F3759DF) - (i >> 1)
    y = plsc.bitcast(i, jnp.float32)
    for _ in range(3):
        y = y * (1.5 - 0.5 * x * y * y)
    return y


def _perm_base(ncols_map):
    lane = _iota16()
    c8 = lane & 7
    blk = jnp.zeros((16,), jnp.int32)
    for c in range(8):
        blk = jnp.where(c8 == c, ncols_map[c], blk)
    return blk * 16 + (lane >> 3)


@functools.partial(
    pl.kernel,
    mesh=_mesh,
    compiler_params=pltpu.CompilerParams(needs_layout_passes=False, use_tc_tiling_on_sc=False),
    out_type=jax.ShapeDtypeStruct((NPAD * 8,), jnp.float32),
    scratch_types=[
        pltpu.VMEM((TN * 4,), jnp.float32),
        pltpu.VMEM((TN,), jnp.int32),
        pltpu.VMEM((Z,), jnp.float32),
        pltpu.VMEM((96,), jnp.float32),
        pltpu.VMEM((TN * 8,), jnp.float32),
    ],
)
def _build_nt1(pos_hbm, num_hbm, rcov_hbm, nt1_hbm, pos_vm, num_vm, rcov_vm,
               stg, nt1_vm):
    wid = lax.axis_index("s") * NC + lax.axis_index("c")
    base = pl.multiple_of(wid * TN, 8)
    pltpu.sync_copy(pos_hbm.at[pl.ds(base * 4, TN * 4)], pos_vm)
    pltpu.sync_copy(num_hbm.at[pl.ds(base, TN)], num_vm)
    pltpu.sync_copy(rcov_hbm, rcov_vm)
    iota = _iota16()
    stg[pl.ds(80, 16)] = jnp.zeros((16,), jnp.float32)
    pbase = _perm_base([0, 1, 2, 3, 4, 5, 5, 5])

    def body(g, carry):
        r4 = (iota + g * 16) * 4
        x = plsc.load_gather(pos_vm, [r4])
        y = plsc.load_gather(pos_vm, [r4 + 1])
        zc = plsc.load_gather(pos_vm, [r4 + 2])
        zn = num_vm[pl.ds(g * 16, 16)]
        rc = plsc.load_gather(rcov_vm, [zn])
        stg[pl.ds(0, 16)] = x
        stg[pl.ds(16, 16)] = y
        stg[pl.ds(32, 16)] = zc
        stg[pl.ds(48, 16)] = rc
        stg[pl.ds(64, 16)] = zn.astype(jnp.float32)
        for k in range(8):
            v = plsc.load_gather(stg, [pbase + 2 * k])
            nt1_vm[pl.ds(g * 128 + k * 16, 16)] = v
        return carry

    lax.fori_loop(0, TN // 16, body, 0)
    pltpu.sync_copy(nt1_vm, nt1_hbm.at[pl.ds(base * 8, TN * 8)])


@functools.partial(
    pl.kernel,
    mesh=_mesh,
    compiler_params=pltpu.CompilerParams(needs_layout_passes=False, use_tc_tiling_on_sc=False),
    out_type=[
        jax.ShapeDtypeStruct((EPAD,), jnp.float32),
        jax.ShapeDtypeStruct((EPAD,), jnp.int32),
        jax.ShapeDtypeStruct((NC * NPAD,), jnp.float32),
    ],
    scratch_types=[
        pltpu.VMEM((ROWS, 128), jnp.int32),
        pltpu.VMEM((CB,), jnp.int32),
        pltpu.VMEM((CB,), jnp.int32),
        pltpu.VMEM((ROWS, 128, 8), jnp.float32),
        pltpu.VMEM((ROWS, 128, 8), jnp.float32),
        pltpu.VMEM((CB,), jnp.float32),
        pltpu.VMEM((CB,), jnp.float32),
        pltpu.VMEM((CB,), jnp.int32),
        pltpu.VMEM((1600,), jnp.float32),
        pltpu.VMEM_SHARED((NPAD,), jnp.float32),
        pltpu.SemaphoreType.DMA,
        pltpu.SemaphoreType.DMA,
    ],
)
def _edge_pass1(srcf_hbm, dstf_hbm, nt1_hbm,
                def_hbm, et_hbm, cnp_hbm,
                srci, srcf, dstf, rowsA, rowsB,
                contrib, defv, etv, zb, shcn, sem, sem2):
    cid = lax.axis_index("c")
    sid = lax.axis_index("s")
    wid = sid * NC + cid
    iota = _iota16()

    def zbody(i, carry):
        zb[pl.ds(i * 16, 16)] = jnp.zeros((16,), jnp.float32)
        return carry

    lax.fori_loop(0, 100, zbody, 0)
    nslice = NPAD // NS

    def zcopy(k, carry):
        pltpu.sync_copy(zb, shcn.at[pl.ds(pl.multiple_of(sid * nslice + k * 1600, 8), 1600)])
        return carry

    lax.fori_loop(0, 4, zcopy, 0)
    plsc.subcore_barrier()

    ebase0 = wid * TE

    def chunk_body(ck, carry):
        ebase = pl.multiple_of(ebase0 + ck * CB, 2048)
        d0 = []
        d0.append(pltpu.async_copy(srcf_hbm.at[pl.ds(ebase, CB)], srcf, sem))
        d0.append(pltpu.async_copy(dstf_hbm.at[pl.ds(ebase, CB)], dstf, sem))
        for j in range(ROWS):
            d0.append(pltpu.async_copy(
                srcf_hbm.at[pl.ds(pl.multiple_of(ebase + j * 128, 128), 128)],
                srci.at[j], sem))
        for dsc in d0:
            dsc.wait()
        descs = []
        for j in range(ROWS):
            descs.append(pltpu.async_copy(
                nt1_hbm.at[srcf.at[pl.ds(j * 128, 128)]], rowsA.at[j], sem))
            descs.append(pltpu.async_copy(
                nt1_hbm.at[dstf.at[pl.ds(j * 128, 128)]], rowsB.at[j], sem))
        for dsc in descs:
            dsc.wait()

        def jbody(j, jcarry):
            jf = _full16(j)
            for l in range(8):
                off = j * 128 + l * 16
                ln = iota + l * 16
                si = srcf[pl.ds(off, 16)]
                di = dstf[pl.ds(off, 16)]
                xi = plsc.load_gather(rowsA, [jf, ln, _full16(0)])
                yi = plsc.load_gather(rowsA, [jf, ln, _full16(1)])
                zi_ = plsc.load_gather(rowsA, [jf, ln, _full16(2)])
                rci = plsc.load_gather(rowsA, [jf, ln, _full16(3)])
                zbi = plsc.load_gather(rowsA, [jf, ln, _full16(4)])
                xj = plsc.load_gather(rowsB, [jf, ln, _full16(0)])
                yj = plsc.load_gather(rowsB, [jf, ln, _full16(1)])
                zj_ = plsc.load_gather(rowsB, [jf, ln, _full16(2)])
                rcj = plsc.load_gather(rowsB, [jf, ln, _full16(3)])
                zbj = plsc.load_gather(rowsB, [jf, ln, _full16(4)])
                dx = xj - xi
                dy = yj - yi
                dz = zj_ - zi_
                dd = dx * dx + dy * dy + dz * dz + 1e-12
                inv = _rsqrt(dd)
                d = dd * inv
                rc = rci + rcj
                cnc = 1.0 / (1.0 + jnp.exp(-K1 * (rc * inv - 1.0)))
                valid = (si != di) & (d > 1e-3)
                contrib[pl.ds(off, 16)] = jnp.where(
                    valid & (d < CN_CUTOFF), cnc, 0.0)
                defv[pl.ds(off, 16)] = jnp.where(valid, d, 1e9)
                zi = zbi.astype(jnp.int32)
                zj = zbj.astype(jnp.int32)
                etv[pl.ds(off, 16)] = zi * Z + zj
            return jcarry

        lax.fori_loop(0, ROWS, jbody, 0)

        d2 = []
        for j in range(ROWS):
            pltpu.async_copy(
                contrib.at[pl.ds(j * 128, 128)], shcn.at[srci.at[j]], sem2,
                add=True).wait()
        d2.append(pltpu.async_copy(defv, def_hbm.at[pl.ds(ebase, CB)], sem2))
        d2.append(pltpu.async_copy(etv, et_hbm.at[pl.ds(ebase, CB)], sem2))
        for dsc in d2:
            dsc.wait()
        return carry

    lax.fori_loop(0, NCHUNK, chunk_body, 0)
    plsc.subcore_barrier()
    pltpu.sync_copy(
        shcn.at[pl.ds(pl.multiple_of(sid * nslice, 8), nslice)],
        cnp_hbm.at[pl.ds(pl.multiple_of(cid * NPAD + sid * nslice, 8), nslice)])


@functools.partial(
    pl.kernel,
    mesh=_mesh,
    compiler_params=pltpu.CompilerParams(needs_layout_passes=False, use_tc_tiling_on_sc=False),
    out_type=jax.ShapeDtypeStruct((NPAD * 8,), jnp.float32),
    scratch_types=[
        pltpu.VMEM((TN,), jnp.float32),
        pltpu.VMEM((TN,), jnp.float32),
        pltpu.VMEM((TN,), jnp.int32),
        pltpu.VMEM((Z * M,), jnp.float32),
        pltpu.VMEM((Z,), jnp.float32),
        pltpu.VMEM((128,), jnp.float32),
        pltpu.VMEM((TN * 8,), jnp.float32),
    ],
)
def _build_nt2(cnp_hbm, num_hbm, cnref_hbm, r4r2_hbm, nt2_hbm,
               cnA, cnB, numv, crv, r4v, stg, nt2_vm):
    wid = lax.axis_index("s") * NC + lax.axis_index("c")
    base = pl.multiple_of(wid * TN, 8)
    pltpu.sync_copy(cnp_hbm.at[pl.ds(base, TN)], cnA)
    pltpu.sync_copy(cnp_hbm.at[pl.ds(pl.multiple_of(NPAD + base, 8), TN)], cnB)
    pltpu.sync_copy(num_hbm.at[pl.ds(base, TN)], numv)
    pltpu.sync_copy(cnref_hbm, crv)
    pltpu.sync_copy(r4r2_hbm, r4v)
    iota = _iota16()
    stg[pl.ds(112, 16)] = jnp.zeros((16,), jnp.float32)
    pbase = _perm_base([0, 1, 2, 3, 4, 5, 6, 7])

    def body(g, carry):
        cn = cnA[pl.ds(g * 16, 16)] + cnB[pl.ds(g * 16, 16)]
        zn = numv[pl.ds(g * 16, 16)]
        zn5 = zn * M
        s = jnp.zeros((16,), jnp.float32)
        ws = []
        for a in range(M):
            cr = plsc.load_gather(crv, [zn5 + a])
            dcn = cn - cr
            w = jnp.exp(-K3 * dcn * dcn)
            w = jnp.where(cr >= 0.0, w, 0.0)
            ws.append(w)
            s = s + w
        inv = jnp.where(s > 1e-30, 1.0 / s, 0.0)
        for a in range(M):
            stg[pl.ds(a * 16, 16)] = ws[a] * inv
        r4 = plsc.load_gather(r4v, [zn])
        stg[pl.ds(80, 16)] = r4
        stg[pl.ds(96, 16)] = s
        for k in range(8):
            v = plsc.load_gather(stg, [pbase + 2 * k])
            nt2_vm[pl.ds(g * 128 + k * 16, 16)] = v
        return carry

    lax.fori_loop(0, TN // 16, body, 0)
    pltpu.sync_copy(nt2_vm, nt2_hbm.at[pl.ds(base * 8, TN * 8)])


@functools.partial(
    pl.kernel,
    mesh=_mesh,
    compiler_params=pltpu.CompilerParams(needs_layout_passes=False, use_tc_tiling_on_sc=False),
    out_type=jax.ShapeDtypeStruct((NW * 16,), jnp.float32),
    scratch_types=[
        pltpu.VMEM((CB,), jnp.int32),
        pltpu.VMEM((CB,), jnp.int32),
        pltpu.VMEM((CB,), jnp.int32),
        pltpu.VMEM((CB,), jnp.float32),
        pltpu.VMEM((ROWS, 128, 8), jnp.float32),
        pltpu.VMEM((ROWS, 128, 8), jnp.float32),
        pltpu.VMEM((ROWS, 128, 32), jnp.float32),
        pltpu.VMEM((16,), jnp.float32),
        pltpu.SemaphoreType.DMA,
    ],
)
def _edge_pass2(srcf_hbm, dstf_hbm, et_hbm, def_hbm, nt2_hbm, c6_hbm, out_hbm,
                srcf, dstf, etf, defv, rowsA, rowsB, c6r, acc, sem):
    cid = lax.axis_index("c")
    sid = lax.axis_index("s")
    wid = sid * NC + cid
    iota = _iota16()
    acc[...] = jnp.zeros((16,), jnp.float32)
    ebase0 = wid * TE

    def chunk_body(ck, carry):
        ebase = pl.multiple_of(ebase0 + ck * CB, 2048)
        d0 = []
        d0.append(pltpu.async_copy(srcf_hbm.at[pl.ds(ebase, CB)], srcf, sem))
        d0.append(pltpu.async_copy(dstf_hbm.at[pl.ds(ebase, CB)], dstf, sem))
        d0.append(pltpu.async_copy(et_hbm.at[pl.ds(ebase, CB)], etf, sem))
        d0.append(pltpu.async_copy(def_hbm.at[pl.ds(ebase, CB)], defv, sem))
        for dsc in d0:
            dsc.wait()
        descs = []
        for j in range(ROWS):
            descs.append(pltpu.async_copy(
                nt2_hbm.at[srcf.at[pl.ds(j * 128, 128)]], rowsA.at[j], sem))
            descs.append(pltpu.async_copy(
                nt2_hbm.at[dstf.at[pl.ds(j * 128, 128)]], rowsB.at[j], sem))
            descs.append(pltpu.async_copy(
                c6_hbm.at[etf.at[pl.ds(j * 128, 128)]], c6r.at[j], sem))
        for dsc in descs:
            dsc.wait()

        def jbody(j, jcarry):
            jf = _full16(j)
            for l in range(8):
                off = j * 128 + l * 16
                ln = iota + l * 16
                d = defv[pl.ds(off, 16)]
                wA = [plsc.load_gather(rowsA, [jf, ln, _full16(a)])
                      for a in range(M)]
                r4i = plsc.load_gather(rowsA, [jf, ln, _full16(5)])
                sA = plsc.load_gather(rowsA, [jf, ln, _full16(6)])
                wB = [plsc.load_gather(rowsB, [jf, ln, _full16(a)])
                      for a in range(M)]
                r4j = plsc.load_gather(rowsB, [jf, ln, _full16(5)])
                sB = plsc.load_gather(rowsB, [jf, ln, _full16(6)])
                c6ij = jnp.zeros((16,), jnp.float32)
                for a in range(M):
                    inner = jnp.zeros((16,), jnp.float32)
                    for b in range(M):
                        cc = plsc.load_gather(c6r, [jf, ln, _full16(a * M + b)])
                        inner = inner + cc * wB[b]
                    c6ij = c6ij + wA[a] * inner
                c6ij = jnp.where(sA * sB > 1e-30, c6ij, 0.0)
                rr = 3.0 * r4i * r4j
                r0 = rr * _rsqrt(rr)
                f = A1 * r0 + A2
                f2 = f * f
                f6 = f2 * f2 * f2
                f8 = f6 * f2
                dsq = d * d
                d6 = dsq * dsq * dsq
                d8 = d6 * dsq
                e = -(S6 * c6ij) / (d6 + f6) - (S8 * (c6ij * rr)) / (d8 + f8)
                e = jnp.where(d < CUTOFF, e, 0.0)
                acc[...] = acc[...] + e
            return jcarry

        lax.fori_loop(0, ROWS, jbody, 0)
        return carry

    lax.fori_loop(0, NCHUNK, chunk_body, 0)
    pltpu.sync_copy(acc, out_hbm.at[pl.ds(pl.multiple_of(wid * 16, 16), 16)])


def kernel(positions, numbers, edge_index, rcov, r4r2, c6, cn_ref):
    positions = positions.astype(jnp.float32)
    numbers = numbers.astype(jnp.int32)
    ei = edge_index.astype(jnp.int32)
    src = jnp.pad(ei[0], (0, EPAD - E))
    dst = jnp.pad(ei[1], (0, EPAD - E))
    pos_p = jnp.pad(positions, ((0, NPAD - N), (0, 1))).reshape(-1)
    num_p = jnp.pad(numbers, (0, NPAD - N))
    c6p = jnp.pad(c6.astype(jnp.float32).reshape(ZZ, M * M), ((0, 0), (0, 7)))

    nt1 = _build_nt1(pos_p, num_p, rcov.astype(jnp.float32))
    d_eff, etype, cnp = _edge_pass1(src, dst, nt1.reshape(NPAD, 8))
    nt2 = _build_nt2(cnp, num_p, cn_ref.astype(jnp.float32).reshape(-1),
                     r4r2.astype(jnp.float32))
    part = _edge_pass2(src, dst, etype, d_eff, nt2.reshape(NPAD, 8), c6p)
    return 0.5 * jnp.sum(part)


# speedup vs baseline: 105.2964x; 1.2975x over previous
import functools

import jax
import jax.numpy as jnp
from jax import lax
from jax.experimental import pallas as pl
from jax.experimental.pallas import tpu as pltpu
from jax.experimental.pallas import tpu_sc as plsc

N = 100000
E = 1600000
Z = 95
M = 5
A1 = 0.4
A2 = 4.8
S6 = 1.0
S8 = 1.0
CUTOFF = 50.0
CN_CUTOFF = 25.0
K1 = 16.0
K3 = 4.0

NC = 2
NS = 16
NW = NC * NS

NPAD = 102400
TN = NPAD // NW
CB = 2048
ROWS = CB // 128
NCHUNK = 25
EPAD = NW * NCHUNK * CB
TE = EPAD // NW
ZZ = Z * Z

_mesh = plsc.VectorSubcoreMesh(core_axis_name="c", subcore_axis_name="s")


def _iota16():
    return lax.iota(jnp.int32, 16)


def _full16(v):
    return jnp.full((16,), v, jnp.int32)


def _rsqrt(x):
    i = plsc.bitcast(x, jnp.int32)
    i = jnp.int32(0x5F3759DF) - (i >> 1)
    y = plsc.bitcast(i, jnp.float32)
    for _ in range(3):
        y = y * (1.5 - 0.5 * x * y * y)
    return y


def _perm_base(ncols_map):
    lane = _iota16()
    c8 = lane & 7
    blk = jnp.zeros((16,), jnp.int32)
    for c in range(8):
        blk = jnp.where(c8 == c, ncols_map[c], blk)
    return blk * 16 + (lane >> 3)


@functools.partial(
    pl.kernel,
    mesh=_mesh,
    compiler_params=pltpu.CompilerParams(needs_layout_passes=False, use_tc_tiling_on_sc=False),
    out_type=jax.ShapeDtypeStruct((NPAD * 8,), jnp.float32),
    scratch_types=[
        pltpu.VMEM((TN * 4,), jnp.float32),
        pltpu.VMEM((TN,), jnp.int32),
        pltpu.VMEM((Z,), jnp.float32),
        pltpu.VMEM((96,), jnp.float32),
        pltpu.VMEM((TN * 8,), jnp.float32),
    ],
)
def _build_nt1(pos_hbm, num_hbm, rcov_hbm, nt1_hbm, pos_vm, num_vm, rcov_vm,
               stg, nt1_vm):
    wid = lax.axis_index("s") * NC + lax.axis_index("c")
    base = pl.multiple_of(wid * TN, 8)
    pltpu.sync_copy(pos_hbm.at[pl.ds(base * 4, TN * 4)], pos_vm)
    pltpu.sync_copy(num_hbm.at[pl.ds(base, TN)], num_vm)
    pltpu.sync_copy(rcov_hbm, rcov_vm)
    iota = _iota16()
    stg[pl.ds(80, 16)] = jnp.zeros((16,), jnp.float32)
    pbase = _perm_base([0, 1, 2, 3, 4, 5, 5, 5])

    def body(g, carry):
        r4 = (iota + g * 16) * 4
        x = plsc.load_gather(pos_vm, [r4])
        y = plsc.load_gather(pos_vm, [r4 + 1])
        zc = plsc.load_gather(pos_vm, [r4 + 2])
        zn = num_vm[pl.ds(g * 16, 16)]
        rc = plsc.load_gather(rcov_vm, [zn])
        stg[pl.ds(0, 16)] = x
        stg[pl.ds(16, 16)] = y
        stg[pl.ds(32, 16)] = zc
        stg[pl.ds(48, 16)] = rc
        stg[pl.ds(64, 16)] = zn.astype(jnp.float32)
        for k in range(8):
            v = plsc.load_gather(stg, [pbase + 2 * k])
            nt1_vm[pl.ds(g * 128 + k * 16, 16)] = v
        return carry

    lax.fori_loop(0, TN // 16, body, 0)
    pltpu.sync_copy(nt1_vm, nt1_hbm.at[pl.ds(base * 8, TN * 8)])


@functools.partial(
    pl.kernel,
    mesh=_mesh,
    compiler_params=pltpu.CompilerParams(needs_layout_passes=False, use_tc_tiling_on_sc=False),
    out_type=[
        jax.ShapeDtypeStruct((EPAD,), jnp.float32),
        jax.ShapeDtypeStruct((EPAD,), jnp.int32),
        jax.ShapeDtypeStruct((NC * NPAD,), jnp.float32),
    ],
    scratch_types=[
        pltpu.VMEM((ROWS, 128), jnp.int32),
        pltpu.VMEM((CB,), jnp.int32),
        pltpu.VMEM((CB,), jnp.int32),
        pltpu.VMEM((ROWS, 128, 8), jnp.float32),
        pltpu.VMEM((ROWS, 128, 8), jnp.float32),
        pltpu.VMEM((CB,), jnp.float32),
        pltpu.VMEM((CB,), jnp.float32),
        pltpu.VMEM((CB,), jnp.int32),
        pltpu.VMEM((1600,), jnp.float32),
        pltpu.VMEM_SHARED((NPAD,), jnp.float32),
        pltpu.SemaphoreType.DMA,
        pltpu.SemaphoreType.DMA,
    ],
)
def _edge_pass1(srcf_hbm, dstf_hbm, nt1_hbm,
                def_hbm, et_hbm, cnp_hbm,
                srci, srcf, dstf, rowsA, rowsB,
                contrib, defv, etv, zb, shcn, sem, sem2):
    cid = lax.axis_index("c")
    sid = lax.axis_index("s")
    wid = sid * NC + cid
    iota = _iota16()

    def zbody(i, carry):
        zb[pl.ds(i * 16, 16)] = jnp.zeros((16,), jnp.float32)
        return carry

    lax.fori_loop(0, 100, zbody, 0)
    nslice = NPAD // NS

    def zcopy(k, carry):
        pltpu.sync_copy(zb, shcn.at[pl.ds(pl.multiple_of(sid * nslice + k * 1600, 8), 1600)])
        return carry

    lax.fori_loop(0, 4, zcopy, 0)
    plsc.subcore_barrier()

    ebase0 = wid * TE

    def chunk_body(ck, carry):
        ebase = pl.multiple_of(ebase0 + ck * CB, 2048)
        d0 = []
        d0.append(pltpu.async_copy(srcf_hbm.at[pl.ds(ebase, CB)], srcf, sem))
        d0.append(pltpu.async_copy(dstf_hbm.at[pl.ds(ebase, CB)], dstf, sem))
        for j in range(ROWS):
            d0.append(pltpu.async_copy(
                srcf_hbm.at[pl.ds(pl.multiple_of(ebase + j * 128, 128), 128)],
                srci.at[j], sem))
        for dsc in d0:
            dsc.wait()
        descs = []
        for j in range(ROWS):
            descs.append(pltpu.async_copy(
                nt1_hbm.at[srcf.at[pl.ds(j * 128, 128)]], rowsA.at[j], sem))
            descs.append(pltpu.async_copy(
                nt1_hbm.at[dstf.at[pl.ds(j * 128, 128)]], rowsB.at[j], sem))
        for dsc in descs:
            dsc.wait()

        def jbody(j, jcarry):
            jf = _full16(j)
            for l in range(8):
                off = j * 128 + l * 16
                ln = iota + l * 16
                si = srcf[pl.ds(off, 16)]
                di = dstf[pl.ds(off, 16)]
                xi = plsc.load_gather(rowsA, [jf, ln, _full16(0)])
                yi = plsc.load_gather(rowsA, [jf, ln, _full16(1)])
                zi_ = plsc.load_gather(rowsA, [jf, ln, _full16(2)])
                rci = plsc.load_gather(rowsA, [jf, ln, _full16(3)])
                zbi = plsc.load_gather(rowsA, [jf, ln, _full16(4)])
                xj = plsc.load_gather(rowsB, [jf, ln, _full16(0)])
                yj = plsc.load_gather(rowsB, [jf, ln, _full16(1)])
                zj_ = plsc.load_gather(rowsB, [jf, ln, _full16(2)])
                rcj = plsc.load_gather(rowsB, [jf, ln, _full16(3)])
                zbj = plsc.load_gather(rowsB, [jf, ln, _full16(4)])
                dx = xj - xi
                dy = yj - yi
                dz = zj_ - zi_
                dd = dx * dx + dy * dy + dz * dz + 1e-12
                inv = _rsqrt(dd)
                d = dd * inv
                rc = rci + rcj
                cnc = 1.0 / (1.0 + jnp.exp(-K1 * (rc * inv - 1.0)))
                valid = (si != di) & (d > 1e-3)
                contrib[pl.ds(off, 16)] = jnp.where(
                    valid & (d < CN_CUTOFF), cnc, 0.0)
                defv[pl.ds(off, 16)] = jnp.where(valid, d, 1e9)
                zi = zbi.astype(jnp.int32)
                zj = zbj.astype(jnp.int32)
                etv[pl.ds(off, 16)] = zi * Z + zj
            return jcarry

        lax.fori_loop(0, ROWS, jbody, 0)

        d2 = []
        pend = []
        for j in range(ROWS):
            pend.append(pltpu.async_copy(
                contrib.at[pl.ds(j * 128, 128)], shcn.at[srci.at[j]], sem2,
                add=True))
            if len(pend) >= 4:
                pend.pop(0).wait()
        for dsc in pend:
            dsc.wait()
        d2.append(pltpu.async_copy(defv, def_hbm.at[pl.ds(ebase, CB)], sem2))
        d2.append(pltpu.async_copy(etv, et_hbm.at[pl.ds(ebase, CB)], sem2))
        for dsc in d2:
            dsc.wait()
        return carry

    lax.fori_loop(0, NCHUNK, chunk_body, 0)
    plsc.subcore_barrier()
    pltpu.sync_copy(
        shcn.at[pl.ds(pl.multiple_of(sid * nslice, 8), nslice)],
        cnp_hbm.at[pl.ds(pl.multiple_of(cid * NPAD + sid * nslice, 8), nslice)])


@functools.partial(
    pl.kernel,
    mesh=_mesh,
    compiler_params=pltpu.CompilerParams(needs_layout_passes=False, use_tc_tiling_on_sc=False),
    out_type=jax.ShapeDtypeStruct((NPAD * 8,), jnp.float32),
    scratch_types=[
        pltpu.VMEM((TN,), jnp.float32),
        pltpu.VMEM((TN,), jnp.float32),
        pltpu.VMEM((TN,), jnp.int32),
        pltpu.VMEM((Z * M,), jnp.float32),
        pltpu.VMEM((Z,), jnp.float32),
        pltpu.VMEM((128,), jnp.float32),
        pltpu.VMEM((TN * 8,), jnp.float32),
    ],
)
def _build_nt2(cnp_hbm, num_hbm, cnref_hbm, r4r2_hbm, nt2_hbm,
               cnA, cnB, numv, crv, r4v, stg, nt2_vm):
    wid = lax.axis_index("s") * NC + lax.axis_index("c")
    base = pl.multiple_of(wid * TN, 8)
    pltpu.sync_copy(cnp_hbm.at[pl.ds(base, TN)], cnA)
    pltpu.sync_copy(cnp_hbm.at[pl.ds(pl.multiple_of(NPAD + base, 8), TN)], cnB)
    pltpu.sync_copy(num_hbm.at[pl.ds(base, TN)], numv)
    pltpu.sync_copy(cnref_hbm, crv)
    pltpu.sync_copy(r4r2_hbm, r4v)
    iota = _iota16()
    stg[pl.ds(112, 16)] = jnp.zeros((16,), jnp.float32)
    pbase = _perm_base([0, 1, 2, 3, 4, 5, 6, 7])

    def body(g, carry):
        cn = cnA[pl.ds(g * 16, 16)] + cnB[pl.ds(g * 16, 16)]
        zn = numv[pl.ds(g * 16, 16)]
        zn5 = zn * M
        s = jnp.zeros((16,), jnp.float32)
        ws = []
        for a in range(M):
            cr = plsc.load_gather(crv, [zn5 + a])
            dcn = cn - cr
            w = jnp.exp(-K3 * dcn * dcn)
            w = jnp.where(cr >= 0.0, w, 0.0)
            ws.append(w)
            s = s + w
        inv = jnp.where(s > 1e-30, 1.0 / s, 0.0)
        for a in range(M):
            stg[pl.ds(a * 16, 16)] = ws[a] * inv
        r4 = plsc.load_gather(r4v, [zn])
        stg[pl.ds(80, 16)] = r4
        stg[pl.ds(96, 16)] = s
        for k in range(8):
            v = plsc.load_gather(stg, [pbase + 2 * k])
            nt2_vm[pl.ds(g * 128 + k * 16, 16)] = v
        return carry

    lax.fori_loop(0, TN // 16, body, 0)
    pltpu.sync_copy(nt2_vm, nt2_hbm.at[pl.ds(base * 8, TN * 8)])


@functools.partial(
    pl.kernel,
    mesh=_mesh,
    compiler_params=pltpu.CompilerParams(needs_layout_passes=False, use_tc_tiling_on_sc=False),
    out_type=jax.ShapeDtypeStruct((NW * 16,), jnp.float32),
    scratch_types=[
        pltpu.VMEM((CB,), jnp.int32),
        pltpu.VMEM((CB,), jnp.int32),
        pltpu.VMEM((CB,), jnp.int32),
        pltpu.VMEM((CB,), jnp.float32),
        pltpu.VMEM((ROWS, 128, 8), jnp.float32),
        pltpu.VMEM((ROWS, 128, 8), jnp.float32),
        pltpu.VMEM((ROWS, 128, 16), jnp.int32),
        pltpu.VMEM((16,), jnp.float32),
        pltpu.SemaphoreType.DMA,
    ],
)
def _edge_pass2(srcf_hbm, dstf_hbm, et_hbm, def_hbm, nt2_hbm, c6_hbm, out_hbm,
                srcf, dstf, etf, defv, rowsA, rowsB, c6r, acc, sem):
    cid = lax.axis_index("c")
    sid = lax.axis_index("s")
    wid = sid * NC + cid
    iota = _iota16()
    acc[...] = jnp.zeros((16,), jnp.float32)
    ebase0 = wid * TE

    def chunk_body(ck, carry):
        ebase = pl.multiple_of(ebase0 + ck * CB, 2048)
        d0 = []
        d0.append(pltpu.async_copy(srcf_hbm.at[pl.ds(ebase, CB)], srcf, sem))
        d0.append(pltpu.async_copy(dstf_hbm.at[pl.ds(ebase, CB)], dstf, sem))
        d0.append(pltpu.async_copy(et_hbm.at[pl.ds(ebase, CB)], etf, sem))
        d0.append(pltpu.async_copy(def_hbm.at[pl.ds(ebase, CB)], defv, sem))
        for dsc in d0:
            dsc.wait()
        descs = []
        for j in range(ROWS):
            descs.append(pltpu.async_copy(
                nt2_hbm.at[srcf.at[pl.ds(j * 128, 128)]], rowsA.at[j], sem))
            descs.append(pltpu.async_copy(
                nt2_hbm.at[dstf.at[pl.ds(j * 128, 128)]], rowsB.at[j], sem))
            descs.append(pltpu.async_copy(
                c6_hbm.at[etf.at[pl.ds(j * 128, 128)]], c6r.at[j], sem))
        for dsc in descs:
            dsc.wait()

        def jbody(j, jcarry):
            jf = _full16(j)
            for l in range(8):
                off = j * 128 + l * 16
                ln = iota + l * 16
                d = defv[pl.ds(off, 16)]
                wA = [plsc.load_gather(rowsA, [jf, ln, _full16(a)])
                      for a in range(M)]
                r4i = plsc.load_gather(rowsA, [jf, ln, _full16(5)])
                sA = plsc.load_gather(rowsA, [jf, ln, _full16(6)])
                wB = [plsc.load_gather(rowsB, [jf, ln, _full16(a)])
                      for a in range(M)]
                r4j = plsc.load_gather(rowsB, [jf, ln, _full16(5)])
                sB = plsc.load_gather(rowsB, [jf, ln, _full16(6)])
                vals = []
                for widx in range(13):
                    word = plsc.load_gather(c6r, [jf, ln, _full16(widx)])
                    vals.append(plsc.bitcast(word << 16, jnp.float32))
                    vals.append(plsc.bitcast(
                        word & jnp.int32(-65536), jnp.float32))
                c6ij = jnp.zeros((16,), jnp.float32)
                for a in range(M):
                    inner = jnp.zeros((16,), jnp.float32)
                    for b in range(M):
                        inner = inner + vals[a * M + b] * wB[b]
                    c6ij = c6ij + wA[a] * inner
                c6ij = jnp.where(sA * sB > 1e-30, c6ij, 0.0)
                rr = 3.0 * r4i * r4j
                r0 = rr * _rsqrt(rr)
                f = A1 * r0 + A2
                f2 = f * f
                f6 = f2 * f2 * f2
                f8 = f6 * f2
                dsq = d * d
                d6 = dsq * dsq * dsq
                d8 = d6 * dsq
                e = -(S6 * c6ij) / (d6 + f6) - (S8 * (c6ij * rr)) / (d8 + f8)
                e = jnp.where(d < CUTOFF, e, 0.0)
                acc[...] = acc[...] + e
            return jcarry

        lax.fori_loop(0, ROWS, jbody, 0)
        return carry

    lax.fori_loop(0, NCHUNK, chunk_body, 0)
    pltpu.sync_copy(acc, out_hbm.at[pl.ds(pl.multiple_of(wid * 16, 16), 16)])


def kernel(positions, numbers, edge_index, rcov, r4r2, c6, cn_ref):
    positions = positions.astype(jnp.float32)
    numbers = numbers.astype(jnp.int32)
    ei = edge_index.astype(jnp.int32)
    src = jnp.pad(ei[0], (0, EPAD - E))
    dst = jnp.pad(ei[1], (0, EPAD - E))
    pos_p = jnp.pad(positions, ((0, NPAD - N), (0, 1))).reshape(-1)
    num_p = jnp.pad(numbers, (0, NPAD - N))
    c6bf = jnp.pad(c6.astype(jnp.bfloat16).reshape(ZZ, M * M),
                   ((0, 0), (0, 7)))
    c6p = jax.lax.bitcast_convert_type(c6bf.reshape(ZZ, 16, 2), jnp.int32)

    nt1 = _build_nt1(pos_p, num_p, rcov.astype(jnp.float32))
    d_eff, etype, cnp = _edge_pass1(src, dst, nt1.reshape(NPAD, 8))
    nt2 = _build_nt2(cnp, num_p, cn_ref.astype(jnp.float32).reshape(-1),
                     r4r2.astype(jnp.float32))
    part = _edge_pass2(src, dst, etype, d_eff, nt2.reshape(NPAD, 8), c6p)
    return 0.5 * jnp.sum(part)


# speedup vs baseline: 105.4174x; 1.0011x over previous
import functools

import jax
import jax.numpy as jnp
from jax import lax
from jax.experimental import pallas as pl
from jax.experimental.pallas import tpu as pltpu
from jax.experimental.pallas import tpu_sc as plsc

N = 100000
E = 1600000
Z = 95
M = 5
A1 = 0.4
A2 = 4.8
S6 = 1.0
S8 = 1.0
CUTOFF = 50.0
CN_CUTOFF = 25.0
K1 = 16.0
K3 = 4.0

NC = 2
NS = 16
NW = NC * NS

NPAD = 102400
TN = NPAD // NW
CB = 2048
ROWS = CB // 128
NCHUNK = 25
EPAD = NW * NCHUNK * CB
TE = EPAD // NW
ZZ = Z * Z

_mesh = plsc.VectorSubcoreMesh(core_axis_name="c", subcore_axis_name="s")
_cparams = pltpu.CompilerParams(
    needs_layout_passes=False, use_tc_tiling_on_sc=False)


def _iota16():
    return lax.iota(jnp.int32, 16)


def _full16(v):
    return jnp.full((16,), v, jnp.int32)


def _rsqrt(x):
    i = plsc.bitcast(x, jnp.int32)
    i = jnp.int32(0x5F3759DF) - (i >> 1)
    y = plsc.bitcast(i, jnp.float32)
    for _ in range(3):
        y = y * (1.5 - 0.5 * x * y * y)
    return y


def _perm_base(ncols_map):
    lane = _iota16()
    c8 = lane & 7
    blk = jnp.zeros((16,), jnp.int32)
    for c in range(8):
        blk = jnp.where(c8 == c, ncols_map[c], blk)
    return blk * 16 + (lane >> 3)


@functools.partial(
    pl.kernel,
    mesh=_mesh,
    compiler_params=_cparams,
    out_type=jax.ShapeDtypeStruct((NPAD * 8,), jnp.float32),
    scratch_types=[
        pltpu.VMEM((TN * 4,), jnp.float32),
        pltpu.VMEM((TN,), jnp.int32),
        pltpu.VMEM((Z,), jnp.float32),
        pltpu.VMEM((96,), jnp.float32),
        pltpu.VMEM((TN * 8,), jnp.float32),
    ],
)
def _build_nt1(pos_hbm, num_hbm, rcov_hbm, nt1_hbm, pos_vm, num_vm, rcov_vm,
               stg, nt1_vm):
    wid = lax.axis_index("s") * NC + lax.axis_index("c")
    base = pl.multiple_of(wid * TN, 8)
    pltpu.sync_copy(pos_hbm.at[pl.ds(base * 4, TN * 4)], pos_vm)
    pltpu.sync_copy(num_hbm.at[pl.ds(base, TN)], num_vm)
    pltpu.sync_copy(rcov_hbm, rcov_vm)
    iota = _iota16()
    stg[pl.ds(80, 16)] = jnp.zeros((16,), jnp.float32)
    pbase = _perm_base([0, 1, 2, 3, 4, 5, 5, 5])

    def body(g, carry):
        r4 = (iota + g * 16) * 4
        x = plsc.load_gather(pos_vm, [r4])
        y = plsc.load_gather(pos_vm, [r4 + 1])
        zc = plsc.load_gather(pos_vm, [r4 + 2])
        zn = num_vm[pl.ds(g * 16, 16)]
        rc = plsc.load_gather(rcov_vm, [zn])
        stg[pl.ds(0, 16)] = x
        stg[pl.ds(16, 16)] = y
        stg[pl.ds(32, 16)] = zc
        stg[pl.ds(48, 16)] = rc
        stg[pl.ds(64, 16)] = zn.astype(jnp.float32)
        for k in range(8):
            v = plsc.load_gather(stg, [pbase + 2 * k])
            nt1_vm[pl.ds(g * 128 + k * 16, 16)] = v
        return carry

    lax.fori_loop(0, TN // 16, body, 0)
    pltpu.sync_copy(nt1_vm, nt1_hbm.at[pl.ds(base * 8, TN * 8)])


@functools.partial(
    pl.kernel,
    mesh=_mesh,
    compiler_params=_cparams,
    out_type=[
        jax.ShapeDtypeStruct((EPAD,), jnp.float32),
        jax.ShapeDtypeStruct((EPAD,), jnp.int32),
        jax.ShapeDtypeStruct((NC * NPAD,), jnp.float32),
    ],
    scratch_types=[
        pltpu.VMEM((ROWS, 128), jnp.int32),
        pltpu.VMEM((CB,), jnp.int32),
        pltpu.VMEM((CB,), jnp.int32),
        pltpu.VMEM((ROWS, 128, 8), jnp.float32),
        pltpu.VMEM((ROWS, 128, 8), jnp.float32),
        pltpu.VMEM((CB,), jnp.float32),
        pltpu.VMEM((CB,), jnp.float32),
        pltpu.VMEM((CB,), jnp.int32),
        pltpu.VMEM((1600,), jnp.float32),
        pltpu.VMEM_SHARED((NPAD,), jnp.float32),
        pltpu.SemaphoreType.DMA,
        pltpu.SemaphoreType.DMA,
    ],
)
def _edge_pass1(srcf_hbm, dstf_hbm, nt1_hbm,
                def_hbm, et_hbm, cnp_hbm,
                srci, srcf, dstf, rowsA, rowsB,
                contrib, defv, etv, zb, shcn, sem, sem2):
    cid = lax.axis_index("c")
    sid = lax.axis_index("s")
    wid = sid * NC + cid
    iota = _iota16()

    def zbody(i, carry):
        zb[pl.ds(i * 16, 16)] = jnp.zeros((16,), jnp.float32)
        return carry

    lax.fori_loop(0, 100, zbody, 0)
    nslice = NPAD // NS

    def zcopy(k, carry):
        pltpu.sync_copy(
            zb,
            shcn.at[pl.ds(pl.multiple_of(sid * nslice + k * 1600, 8), 1600)])
        return carry

    lax.fori_loop(0, 4, zcopy, 0)
    plsc.subcore_barrier()

    ebase0 = wid * TE

    def chunk_body(ck, carry):
        ebase = pl.multiple_of(ebase0 + ck * CB, 2048)
        d0 = []
        d0.append(pltpu.async_copy(srcf_hbm.at[pl.ds(ebase, CB)], srcf, sem))
        d0.append(pltpu.async_copy(dstf_hbm.at[pl.ds(ebase, CB)], dstf, sem))
        for j in range(ROWS):
            d0.append(pltpu.async_copy(
                srcf_hbm.at[pl.ds(pl.multiple_of(ebase + j * 128, 128), 128)],
                srci.at[j], sem))
        for dsc in d0:
            dsc.wait()
        descs = []
        for j in range(ROWS):
            descs.append(pltpu.async_copy(
                nt1_hbm.at[srcf.at[pl.ds(j * 128, 128)]], rowsA.at[j], sem))
            descs.append(pltpu.async_copy(
                nt1_hbm.at[dstf.at[pl.ds(j * 128, 128)]], rowsB.at[j], sem))
        for dsc in descs:
            dsc.wait()

        def jbody(j, jcarry):
            jf = _full16(j)
            for l in range(8):
                off = j * 128 + l * 16
                ln = iota + l * 16
                si = srcf[pl.ds(off, 16)]
                di = dstf[pl.ds(off, 16)]
                xi = plsc.load_gather(rowsA, [jf, ln, _full16(0)])
                yi = plsc.load_gather(rowsA, [jf, ln, _full16(1)])
                zi_ = plsc.load_gather(rowsA, [jf, ln, _full16(2)])
                rci = plsc.load_gather(rowsA, [jf, ln, _full16(3)])
                zbi = plsc.load_gather(rowsA, [jf, ln, _full16(4)])
                xj = plsc.load_gather(rowsB, [jf, ln, _full16(0)])
                yj = plsc.load_gather(rowsB, [jf, ln, _full16(1)])
                zj_ = plsc.load_gather(rowsB, [jf, ln, _full16(2)])
                rcj = plsc.load_gather(rowsB, [jf, ln, _full16(3)])
                zbj = plsc.load_gather(rowsB, [jf, ln, _full16(4)])
                dx = xj - xi
                dy = yj - yi
                dz = zj_ - zi_
                dd = dx * dx + dy * dy + dz * dz + 1e-12
                inv = _rsqrt(dd)
                d = dd * inv
                rc = rci + rcj
                cnc = 1.0 / (1.0 + jnp.exp(-K1 * (rc * inv - 1.0)))
                valid = (si != di) & (d > 1e-3)
                contrib[pl.ds(off, 16)] = jnp.where(
                    valid & (d < CN_CUTOFF), cnc, 0.0)
                defv[pl.ds(off, 16)] = jnp.where(valid, d, 1e9)
                zi = zbi.astype(jnp.int32)
                zj = zbj.astype(jnp.int32)
                etv[pl.ds(off, 16)] = zi * Z + zj
            return jcarry

        lax.fori_loop(0, ROWS, jbody, 0)

        pend = []
        for j in range(ROWS):
            pend.append(pltpu.async_copy(
                contrib.at[pl.ds(j * 128, 128)], shcn.at[srci.at[j]], sem2,
                add=True))
            if len(pend) >= 4:
                pend.pop(0).wait()
        for dsc in pend:
            dsc.wait()
        d2 = []
        d2.append(pltpu.async_copy(defv, def_hbm.at[pl.ds(ebase, CB)], sem2))
        d2.append(pltpu.async_copy(etv, et_hbm.at[pl.ds(ebase, CB)], sem2))
        for dsc in d2:
            dsc.wait()
        return carry

    lax.fori_loop(0, NCHUNK, chunk_body, 0)
    plsc.subcore_barrier()
    pltpu.sync_copy(
        shcn.at[pl.ds(pl.multiple_of(sid * nslice, 8), nslice)],
        cnp_hbm.at[pl.ds(pl.multiple_of(cid * NPAD + sid * nslice, 8),
                         nslice)])


@functools.partial(
    pl.kernel,
    mesh=_mesh,
    compiler_params=_cparams,
    out_type=jax.ShapeDtypeStruct((NPAD * 8,), jnp.float32),
    scratch_types=[
        pltpu.VMEM((TN,), jnp.float32),
        pltpu.VMEM((TN,), jnp.float32),
        pltpu.VMEM((TN,), jnp.int32),
        pltpu.VMEM((Z * M,), jnp.float32),
        pltpu.VMEM((Z,), jnp.float32),
        pltpu.VMEM((128,), jnp.float32),
        pltpu.VMEM((TN * 8,), jnp.float32),
    ],
)
def _build_nt2(cnp_hbm, num_hbm, cnref_hbm, r4r2_hbm, nt2_hbm,
               cnA, cnB, numv, crv, r4v, stg, nt2_vm):
    wid = lax.axis_index("s") * NC + lax.axis_index("c")
    base = pl.multiple_of(wid * TN, 8)
    pltpu.sync_copy(cnp_hbm.at[pl.ds(base, TN)], cnA)
    pltpu.sync_copy(cnp_hbm.at[pl.ds(pl.multiple_of(NPAD + base, 8), TN)], cnB)
    pltpu.sync_copy(num_hbm.at[pl.ds(base, TN)], numv)
    pltpu.sync_copy(cnref_hbm, crv)
    pltpu.sync_copy(r4r2_hbm, r4v)
    iota = _iota16()
    stg[pl.ds(112, 16)] = jnp.zeros((16,), jnp.float32)
    pbase = _perm_base([0, 1, 2, 3, 4, 5, 6, 7])

    def body(g, carry):
        cn = cnA[pl.ds(g * 16, 16)] + cnB[pl.ds(g * 16, 16)]
        zn = numv[pl.ds(g * 16, 16)]
        zn5 = zn * M
        s = jnp.zeros((16,), jnp.float32)
        ws = []
        for a in range(M):
            cr = plsc.load_gather(crv, [zn5 + a])
            dcn = cn - cr
            w = jnp.exp(-K3 * dcn * dcn)
            w = jnp.where(cr >= 0.0, w, 0.0)
            ws.append(w)
            s = s + w
        inv = jnp.where(s > 1e-30, 1.0 / s, 0.0)
        for a in range(M):
            stg[pl.ds(a * 16, 16)] = ws[a] * inv
        r4 = plsc.load_gather(r4v, [zn])
        stg[pl.ds(80, 16)] = r4
        stg[pl.ds(96, 16)] = s
        for k in range(8):
            v = plsc.load_gather(stg, [pbase + 2 * k])
            nt2_vm[pl.ds(g * 128 + k * 16, 16)] = v
        return carry

    lax.fori_loop(0, TN // 16, body, 0)
    pltpu.sync_copy(nt2_vm, nt2_hbm.at[pl.ds(base * 8, TN * 8)])


@functools.partial(
    pl.kernel,
    mesh=_mesh,
    compiler_params=_cparams,
    out_type=jax.ShapeDtypeStruct((NW * 16,), jnp.float32),
    scratch_types=[
        pltpu.VMEM((CB,), jnp.int32),
        pltpu.VMEM((CB,), jnp.int32),
        pltpu.VMEM((CB,), jnp.int32),
        pltpu.VMEM((CB,), jnp.float32),
        pltpu.VMEM((ROWS, 128, 8), jnp.float32),
        pltpu.VMEM((ROWS, 128, 8), jnp.float32),
        pltpu.VMEM((ROWS, 128, 16), jnp.int32),
        pltpu.VMEM((16,), jnp.float32),
        pltpu.SemaphoreType.DMA,
    ],
)
def _edge_pass2(srcf_hbm, dstf_hbm, et_hbm, def_hbm, nt2_hbm, c6_hbm, out_hbm,
                srcf, dstf, etf, defv, rowsA, rowsB, c6r, acc, sem):
    cid = lax.axis_index("c")
    sid = lax.axis_index("s")
    wid = sid * NC + cid
    iota = _iota16()
    acc[...] = jnp.zeros((16,), jnp.float32)
    ebase0 = wid * TE

    def chunk_body(ck, carry):
        ebase = pl.multiple_of(ebase0 + ck * CB, 2048)
        d0 = []
        d0.append(pltpu.async_copy(srcf_hbm.at[pl.ds(ebase, CB)], srcf, sem))
        d0.append(pltpu.async_copy(dstf_hbm.at[pl.ds(ebase, CB)], dstf, sem))
        d0.append(pltpu.async_copy(et_hbm.at[pl.ds(ebase, CB)], etf, sem))
        d0.append(pltpu.async_copy(def_hbm.at[pl.ds(ebase, CB)], defv, sem))
        for dsc in d0:
            dsc.wait()
        descs = []
        for j in range(ROWS):
            descs.append(pltpu.async_copy(
                nt2_hbm.at[srcf.at[pl.ds(j * 128, 128)]], rowsA.at[j], sem))
            descs.append(pltpu.async_copy(
                nt2_hbm.at[dstf.at[pl.ds(j * 128, 128)]], rowsB.at[j], sem))
            descs.append(pltpu.async_copy(
                c6_hbm.at[etf.at[pl.ds(j * 128, 128)]], c6r.at[j], sem))
        for dsc in descs:
            dsc.wait()

        def jbody(j, jcarry):
            jf = _full16(j)
            for l in range(8):
                off = j * 128 + l * 16
                ln = iota + l * 16
                d = defv[pl.ds(off, 16)]
                wA = [plsc.load_gather(rowsA, [jf, ln, _full16(a)])
                      for a in range(M)]
                r4i = plsc.load_gather(rowsA, [jf, ln, _full16(5)])
                sA = plsc.load_gather(rowsA, [jf, ln, _full16(6)])
                wB = [plsc.load_gather(rowsB, [jf, ln, _full16(a)])
                      for a in range(M)]
                r4j = plsc.load_gather(rowsB, [jf, ln, _full16(5)])
                sB = plsc.load_gather(rowsB, [jf, ln, _full16(6)])
                vals = []
                for widx in range(13):
                    word = plsc.load_gather(c6r, [jf, ln, _full16(widx)])
                    vals.append(plsc.bitcast(word << 16, jnp.float32))
                    vals.append(plsc.bitcast(
                        word & jnp.int32(-65536), jnp.float32))
                c6ij = jnp.zeros((16,), jnp.float32)
                for a in range(M):
                    inner = jnp.zeros((16,), jnp.float32)
                    for b in range(M):
                        inner = inner + vals[a * M + b] * wB[b]
                    c6ij = c6ij + wA[a] * inner
                c6ij = jnp.where(sA * sB > 1e-30, c6ij, 0.0)
                rr = 3.0 * r4i * r4j
                r0 = rr * _rsqrt(rr)
                f = A1 * r0 + A2
                f2 = f * f
                f6 = f2 * f2 * f2
                f8 = f6 * f2
                dsq = d * d
                d6 = dsq * dsq * dsq
                d8 = d6 * dsq
                e = -(S6 * c6ij) / (d6 + f6) - (S8 * (c6ij * rr)) / (d8 + f8)
                e = jnp.where(d < CUTOFF, e, 0.0)
                acc[...] = acc[...] + e
            return jcarry

        lax.fori_loop(0, ROWS, jbody, 0)
        return carry

    lax.fori_loop(0, NCHUNK, chunk_body, 0)
    pltpu.sync_copy(acc, out_hbm.at[pl.ds(pl.multiple_of(wid * 16, 16), 16)])


def kernel(positions, numbers, edge_index, rcov, r4r2, c6, cn_ref):
    positions = positions.astype(jnp.float32)
    numbers = numbers.astype(jnp.int32)
    ei = edge_index.astype(jnp.int32)
    src = jnp.pad(ei[0], (0, EPAD - E))
    dst = jnp.pad(ei[1], (0, EPAD - E))
    pos_p = jnp.pad(positions, ((0, NPAD - N), (0, 1))).reshape(-1)
    num_p = jnp.pad(numbers, (0, NPAD - N))
    c6bf = jnp.pad(c6.astype(jnp.bfloat16).reshape(ZZ, M * M),
                   ((0, 0), (0, 7)))
    c6p = jax.lax.bitcast_convert_type(c6bf.reshape(ZZ, 16, 2), jnp.int32)

    nt1 = _build_nt1(pos_p, num_p, rcov.astype(jnp.float32))
    d_eff, etype, cnp = _edge_pass1(src, dst, nt1.reshape(NPAD, 8))
    nt2 = _build_nt2(cnp, num_p, cn_ref.astype(jnp.float32).reshape(-1),
                     r4r2.astype(jnp.float32))
    part = _edge_pass2(src, dst, etype, d_eff, nt2.reshape(NPAD, 8), c6p)
    return 0.5 * jnp.sum(part)
